# Initial kernel scaffold; baseline (speedup 1.0000x reference)
import jax
import jax.numpy as jnp
from jax.experimental import pallas as pl


def kernel(x, edge_index_local, edge_index_global, lW0, lb0, lW1, lb1, lg0, lbe0, lg1, lbe1, gW0, gb0, gW1, gb1, gg0, gbe0, gg1, gbe1, mW0, mb0, mW1, mb1, cW0, cb0, cW1, cb1):
    raise NotImplementedError("write your pallas kernel here")


# speedup vs baseline: 3.6694x; 3.6694x over previous
import functools

import jax
import jax.numpy as jnp
from jax import lax
from jax.experimental import pallas as pl
from jax.experimental.pallas import tpu as pltpu
from jax.experimental.pallas import tpu_sc as plsc

N_PAD = 10240
BLK = 1024
NW = 32
CHUNK = 128
BN_S = float(1.0 / (1.0 + 1e-5) ** 0.5)

_MESH = dict(core_axis_name="c", subcore_axis_name="s")


def _sc_seg_sum(table, srcw, dstw, k):
    rpt = N_PAD // 16

    @functools.partial(
        pl.kernel,
        out_type=jax.ShapeDtypeStruct((2 * N_PAD, 128), jnp.float32),
        mesh=plsc.VectorSubcoreMesh(**_MESH),
        scratch_types=[
            pltpu.VMEM((k, CHUNK), jnp.int32),
            pltpu.VMEM((k, CHUNK), jnp.int32),
            pltpu.VMEM((CHUNK, 128), jnp.float32),
            pltpu.VMEM_SHARED((N_PAD, 128), jnp.float32),
        ],
    )
    def seg(table_hbm, src_hbm, dst_hbm, out_hbm, src_v, dst_v, buf_v,
            acc_sh):
        cid = lax.axis_index("c")
        sid = lax.axis_index("s")
        wid = cid * 16 + sid

        @pl.loop(0, CHUNK)
        def _(r):
            for cc in range(8):
                buf_v[r, pl.ds(cc * 16, 16)] = jnp.zeros((16,), jnp.float32)

        base = sid * rpt
        for jj in range(rpt // 128):
            pltpu.sync_copy(buf_v, acc_sh.at[pl.ds(base + jj * 128, 128)])
        plsc.subcore_barrier()

        pltpu.sync_copy(src_hbm.at[pl.ds(wid * k, k)], src_v)
        pltpu.sync_copy(dst_hbm.at[pl.ds(wid * k, k)], dst_v)

        @pl.loop(0, k)
        def _(j):
            pltpu.sync_copy(table_hbm.at[src_v.at[j]], buf_v)
            pltpu.sync_copy(buf_v, acc_sh.at[dst_v.at[j]], add=True)

        plsc.subcore_barrier()
        for jj in range(rpt // 128):
            pltpu.sync_copy(
                acc_sh.at[pl.ds(base + jj * 128, 128)],
                out_hbm.at[pl.ds(cid * N_PAD + base + jj * 128, 128)])

    return seg(table, srcw, dstw)


def _dis_of(degp):
    deg = degp[0] + degp[1]
    return jnp.where(deg > 0.0, lax.rsqrt(jnp.maximum(deg, 1e-30)),
                     0.0)[:, 0:1]


def _full(shape):
    return pl.BlockSpec(shape, lambda *_: tuple(0 for _ in shape))


def _t1_body(x_ref, w_ref, degp_ref, lo_ref, hi_ref):
    h = jnp.dot(x_ref[...], w_ref[...], preferred_element_type=jnp.float32)
    dis = _dis_of(degp_ref[...])
    hs = h * dis
    lo_ref[...] = hs[:, :128]
    hi_ref[...] = hs[:, 128:]


def _tc_scale_mm(x, w, degp):
    grid = (N_PAD // BLK,)
    out = pl.pallas_call(
        _t1_body,
        grid=grid,
        in_specs=[
            pl.BlockSpec((BLK, 128), lambda i: (i, 0)),
            _full(w.shape),
            pl.BlockSpec((2, BLK, 128), lambda i: (0, i, 0)),
        ],
        out_specs=[pl.BlockSpec((BLK, 128), lambda i: (i, 0))] * 2,
        out_shape=[jax.ShapeDtypeStruct((N_PAD, 128), jnp.float32)] * 2,
    )(x, w, degp)
    return out[0], out[1]


def _t2_body(alo_ref, ahi_ref, degp_ref, b0_ref, g0_ref, be0_ref, w1_ref,
             out_ref):
    alo = alo_ref[...]
    ahi = ahi_ref[...]
    h0 = jnp.concatenate([alo[0] + alo[1], ahi[0] + ahi[1]], axis=1)
    dis = _dis_of(degp_ref[...])
    z = jnp.maximum(dis * h0 + b0_ref[...], 0.0)
    zb = z * (g0_ref[...] * BN_S) + be0_ref[...]
    h1 = jnp.dot(zb, w1_ref[...], preferred_element_type=jnp.float32)
    out_ref[...] = h1 * dis


def _tc_mid(alo, ahi, degp, b0, g0, be0, w1):
    grid = (N_PAD // BLK,)
    return pl.pallas_call(
        _t2_body,
        grid=grid,
        in_specs=[
            pl.BlockSpec((2, BLK, 128), lambda i: (0, i, 0)),
            pl.BlockSpec((2, BLK, 128), lambda i: (0, i, 0)),
            pl.BlockSpec((2, BLK, 128), lambda i: (0, i, 0)),
            _full((1, 256)), _full((1, 256)), _full((1, 256)),
            _full((256, 128)),
        ],
        out_specs=pl.BlockSpec((BLK, 128), lambda i: (i, 0)),
        out_shape=jax.ShapeDtypeStruct((N_PAD, 128), jnp.float32),
    )(alo, ahi, degp, b0, g0, be0, w1)


def _t3_body(a_ref, degp_ref, b1_ref, g1_ref, be1_ref, out_ref):
    a = a_ref[...]
    dis = _dis_of(degp_ref[...])
    h = dis * (a[0] + a[1]) + b1_ref[...]
    out_ref[...] = h * (g1_ref[...] * BN_S) + be1_ref[...]


def _tc_post(acc, degp, b1, g1, be1):
    grid = (N_PAD // BLK,)
    return pl.pallas_call(
        _t3_body,
        grid=grid,
        in_specs=[
            pl.BlockSpec((2, BLK, 128), lambda i: (0, i, 0)),
            pl.BlockSpec((2, BLK, 128), lambda i: (0, i, 0)),
            _full((1, 128)), _full((1, 128)), _full((1, 128)),
        ],
        out_specs=pl.BlockSpec((BLK, 128), lambda i: (i, 0)),
        out_shape=jax.ShapeDtypeStruct((N_PAD, 128), jnp.float32),
    )(acc, degp, b1, g1, be1)


def _t4_body(n_valid, hl_ref, hg_ref, mw0_ref, mb0_ref, mw1_ref, mb1_ref,
             cw0_ref, cb0_ref, cw1_ref, cb1_ref, zl_ref, zg_ref, pred_ref,
             sl_ref, sg_ref):
    i = pl.program_id(0)

    def mlp(h):
        a = jnp.maximum(
            jnp.dot(h, mw0_ref[...], preferred_element_type=jnp.float32)
            + mb0_ref[...], 0.0)
        return jnp.dot(a, mw1_ref[...],
                       preferred_element_type=jnp.float32) + mb1_ref[...]

    zl = mlp(hl_ref[...])
    zg = mlp(hg_ref[...])
    zl_ref[...] = zl
    zg_ref[...] = zg
    s = zl + zg
    a = jnp.maximum(
        jnp.dot(s, cw0_ref[...], preferred_element_type=jnp.float32)
        + cb0_ref[...], 0.0)
    pred_ref[...] = jnp.dot(a, cw1_ref[...],
                            preferred_element_type=jnp.float32) + cb1_ref[...]

    rows = lax.broadcasted_iota(jnp.int32, (BLK, 1), 0) + i * BLK
    m = (rows < n_valid).astype(jnp.float32)

    @pl.when(i == 0)
    def _():
        sl_ref[...] = jnp.zeros_like(sl_ref)
        sg_ref[...] = jnp.zeros_like(sg_ref)

    sl_ref[...] += jnp.sum(zl * m, axis=0, keepdims=True)
    sg_ref[...] += jnp.sum(zg * m, axis=0, keepdims=True)


def _tc_heads(n_valid, hl, hg, mw0, mb0, mw1, mb1, cw0, cb0, cw1, cb1):
    grid = (N_PAD // BLK,)
    return pl.pallas_call(
        functools.partial(_t4_body, n_valid),
        grid=grid,
        in_specs=[
            pl.BlockSpec((BLK, 128), lambda i: (i, 0)),
            pl.BlockSpec((BLK, 128), lambda i: (i, 0)),
            _full((128, 128)), _full((1, 128)),
            _full((128, 128)), _full((1, 128)),
            _full((128, 64)), _full((1, 64)),
            _full((64, 16)), _full((1, 16)),
        ],
        out_specs=[
            pl.BlockSpec((BLK, 128), lambda i: (i, 0)),
            pl.BlockSpec((BLK, 128), lambda i: (i, 0)),
            pl.BlockSpec((BLK, 16), lambda i: (i, 0)),
            pl.BlockSpec((1, 128), lambda i: (0, 0)),
            pl.BlockSpec((1, 128), lambda i: (0, 0)),
        ],
        out_shape=[
            jax.ShapeDtypeStruct((N_PAD, 128), jnp.float32),
            jax.ShapeDtypeStruct((N_PAD, 128), jnp.float32),
            jax.ShapeDtypeStruct((N_PAD, 16), jnp.float32),
            jax.ShapeDtypeStruct((1, 128), jnp.float32),
            jax.ShapeDtypeStruct((1, 128), jnp.float32),
        ],
    )(hl, hg, mw0, mb0, mw1, mb1, cw0, cb0, cw1, cb1)


def _t5_body(s_ref, mw0_ref, mb0_ref, mw1_ref, mb1_ref, out_ref):
    a = jnp.maximum(
        jnp.dot(s_ref[...], mw0_ref[...], preferred_element_type=jnp.float32)
        + mb0_ref[...], 0.0)
    out_ref[...] = jnp.dot(a, mw1_ref[...],
                           preferred_element_type=jnp.float32) + mb1_ref[...]


def _tc_graph_mlp(s, mw0, mb0, mw1, mb1):
    return pl.pallas_call(
        _t5_body,
        in_specs=[_full((8, 128)), _full((128, 128)), _full((1, 128)),
                  _full((128, 128)), _full((1, 128))],
        out_specs=_full((8, 128)),
        out_shape=jax.ShapeDtypeStruct((8, 128), jnp.float32),
    )(s, mw0, mb0, mw1, mb1)


def _prep_edges(ei, k):
    e = ei.shape[1]
    e_pad = NW * CHUNK * k
    src = jnp.concatenate(
        [ei[0], jnp.zeros((e_pad - e,), jnp.int32)]).reshape(NW * k, CHUNK)
    dst = jnp.concatenate(
        [ei[1], jnp.full((e_pad - e,), N_PAD - 1, jnp.int32)]
    ).reshape(NW * k, CHUNK)
    return src, dst


def kernel(x, edge_index_local, edge_index_global, lW0, lb0, lW1, lb1, lg0,
           lbe0, lg1, lbe1, gW0, gb0, gW1, gb1, gg0, gbe0, gg1, gbe1, mW0,
           mb0, mW1, mb1, cW0, cb0, cW1, cb1):
    n = x.shape[0]
    e = edge_index_local.shape[1]
    k = -(-e // (NW * CHUNK))
    k = -(-k // 8) * 8

    x_pad = jnp.pad(x, ((0, N_PAD - n), (0, 0)))
    ones_tab = jnp.ones((N_PAD, 128), jnp.float32)
    row = lambda v: v.reshape(1, -1)

    outs = {}
    for p, ei, W0, b0, W1, b1, g0, be0, g1, be1 in (
        ("l", edge_index_local, lW0, lb0, lW1, lb1, lg0, lbe0, lg1, lbe1),
        ("g", edge_index_global, gW0, gb0, gW1, gb1, gg0, gbe0, gg1, gbe1),
    ):
        srcw, dstw = _prep_edges(ei, k)
        degp = _sc_seg_sum(ones_tab, srcw, dstw, k).reshape(2, N_PAD, 128)
        hs_lo, hs_hi = _tc_scale_mm(x_pad, W0, degp)
        a_lo = _sc_seg_sum(hs_lo, srcw, dstw, k).reshape(2, N_PAD, 128)
        a_hi = _sc_seg_sum(hs_hi, srcw, dstw, k).reshape(2, N_PAD, 128)
        hs1 = _tc_mid(a_lo, a_hi, degp, row(b0), row(g0), row(be0), W1)
        a1 = _sc_seg_sum(hs1, srcw, dstw, k).reshape(2, N_PAD, 128)
        outs[p] = _tc_post(a1, degp, row(b1), row(g1), row(be1))

    zl, zg, pred, sl, sg = _tc_heads(
        n, outs["l"], outs["g"], mW0, row(mb0), mW1, row(mb1), cW0, row(cb0),
        cW1, row(cb1))
    s8 = jnp.concatenate([sl, sg, jnp.zeros((6, 128), jnp.float32)], axis=0)
    gs = _tc_graph_mlp(s8, mW0, row(mb0), mW1, row(mb1))
    return (zl[:n], zg[:n], gs[0:1], gs[1:2], pred[:n])


# speedup vs baseline: 5.0076x; 1.3647x over previous
import functools

import jax
import jax.numpy as jnp
from jax import lax
from jax.experimental import pallas as pl
from jax.experimental.pallas import tpu as pltpu
from jax.experimental.pallas import tpu_sc as plsc

N_PAD = 10240
BLK = 1024
NW = 32
CHUNK = 128
BN_S = float(1.0 / (1.0 + 1e-5) ** 0.5)

_MESH = dict(core_axis_name="c", subcore_axis_name="s")


def _fill(ref, val):
    @pl.loop(0, CHUNK)
    def _(r):
        for cc in range(8):
            ref[r, pl.ds(cc * 16, 16)] = jnp.full((16,), val, jnp.float32)


def _zero_acc(buf_v, acc_sh, base):
    _fill(buf_v, 0.0)
    for jj in range(N_PAD // 16 // 128):
        pltpu.sync_copy(buf_v, acc_sh.at[pl.ds(base + jj * 128, 128)])


def _write_out(acc_sh, out_hbm, cid, base):
    for jj in range(N_PAD // 16 // 128):
        pltpu.sync_copy(
            acc_sh.at[pl.ds(base + jj * 128, 128)],
            out_hbm.at[pl.ds(cid * N_PAD + base + jj * 128, 128)])


def _sc_seg_sum(table, srcw, dstw, k):
    rpt = N_PAD // 16
    kh = k // 2

    @functools.partial(
        pl.kernel,
        out_type=jax.ShapeDtypeStruct((2 * N_PAD, 128), jnp.float32),
        mesh=plsc.VectorSubcoreMesh(**_MESH),
        scratch_types=[
            pltpu.VMEM((kh, CHUNK), jnp.int32),
            pltpu.VMEM((kh, CHUNK), jnp.int32),
            pltpu.VMEM((CHUNK, 128), jnp.float32),
            pltpu.VMEM((CHUNK, 128), jnp.float32),
            pltpu.VMEM_SHARED((N_PAD, 128), jnp.float32),
            pltpu.SemaphoreType.DMA,
            pltpu.SemaphoreType.DMA,
            pltpu.SemaphoreType.DMA,
            pltpu.SemaphoreType.DMA,
        ],
    )
    def seg(table_hbm, src_hbm, dst_hbm, out_hbm, src_v, dst_v, buf0, buf1,
            acc_sh, g0, g1, s0, s1):
        cid = lax.axis_index("c")
        sid = lax.axis_index("s")
        wid = cid * 16 + sid
        bufs, gsem, ssem = (buf0, buf1), (g0, g1), (s0, s1)

        def wait(b, sem):
            pltpu.make_async_copy(table_hbm.at[pl.ds(0, CHUNK)], bufs[b],
                                  sem[b]).wait()

        base = sid * rpt
        _zero_acc(buf0, acc_sh, base)
        plsc.subcore_barrier()

        for ph in range(2):
            off = wid * k + ph * kh
            pltpu.sync_copy(src_hbm.at[pl.ds(off, kh)], src_v)
            pltpu.sync_copy(dst_hbm.at[pl.ds(off, kh)], dst_v)
            for b in range(2):
                pltpu.async_copy(table_hbm.at[src_v.at[b]], bufs[b], gsem[b])

            @pl.loop(0, kh - 2, step=2)
            def _(j):
                for b in range(2):
                    jj = j + b
                    wait(b, gsem)
                    pltpu.async_copy(bufs[b], acc_sh.at[dst_v.at[jj]],
                                     ssem[b], add=True)
                    wait(b, ssem)
                    pltpu.async_copy(table_hbm.at[src_v.at[jj + 2]], bufs[b],
                                     gsem[b])

            for b in range(2):
                wait(b, gsem)
                pltpu.sync_copy(bufs[b], acc_sh.at[dst_v.at[kh - 2 + b]],
                                add=True)

        plsc.subcore_barrier()
        _write_out(acc_sh, out_hbm, cid, base)

    return seg(table, srcw, dstw)


def _sc_deg(dstw, k):
    rpt = N_PAD // 16

    @functools.partial(
        pl.kernel,
        out_type=jax.ShapeDtypeStruct((2 * N_PAD, 128), jnp.float32),
        mesh=plsc.VectorSubcoreMesh(**_MESH),
        scratch_types=[
            pltpu.VMEM((k, CHUNK), jnp.int32),
            pltpu.VMEM((CHUNK, 128), jnp.float32),
            pltpu.VMEM_SHARED((N_PAD, 128), jnp.float32),
            pltpu.SemaphoreType.DMA,
        ],
    )
    def deg(dst_hbm, out_hbm, dst_v, ones_v, acc_sh, sem):
        cid = lax.axis_index("c")
        sid = lax.axis_index("s")
        wid = cid * 16 + sid
        base = sid * rpt

        _zero_acc(ones_v, acc_sh, base)
        _fill(ones_v, 1.0)
        plsc.subcore_barrier()

        pltpu.sync_copy(dst_hbm.at[pl.ds(wid * k, k)], dst_v)

        @pl.loop(0, k, step=8)
        def _(j):
            for b in range(8):
                pltpu.async_copy(ones_v, acc_sh.at[dst_v.at[j + b]], sem,
                                 add=True)
            for b in range(8):
                pltpu.make_async_copy(out_hbm.at[pl.ds(0, CHUNK)], ones_v,
                                      sem).wait()

        plsc.subcore_barrier()
        _write_out(acc_sh, out_hbm, cid, base)

    return deg(dstw)


def _dis_of(degp):
    deg = degp[0] + degp[1]
    return jnp.where(deg > 0.0, lax.rsqrt(jnp.maximum(deg, 1e-30)),
                     0.0)[:, 0:1]


def _full(shape):
    return pl.BlockSpec(shape, lambda *_: tuple(0 for _ in shape))


def _t1_body(x_ref, w_ref, degp_ref, lo_ref, hi_ref):
    h = jnp.dot(x_ref[...], w_ref[...], preferred_element_type=jnp.float32)
    dis = _dis_of(degp_ref[...])
    hs = h * dis
    lo_ref[...] = hs[:, :128]
    hi_ref[...] = hs[:, 128:]


def _tc_scale_mm(x, w, degp):
    grid = (N_PAD // BLK,)
    out = pl.pallas_call(
        _t1_body,
        grid=grid,
        in_specs=[
            pl.BlockSpec((BLK, 128), lambda i: (i, 0)),
            _full(w.shape),
            pl.BlockSpec((2, BLK, 128), lambda i: (0, i, 0)),
        ],
        out_specs=[pl.BlockSpec((BLK, 128), lambda i: (i, 0))] * 2,
        out_shape=[jax.ShapeDtypeStruct((N_PAD, 128), jnp.float32)] * 2,
    )(x, w, degp)
    return out[0], out[1]


def _t2_body(alo_ref, ahi_ref, degp_ref, b0_ref, g0_ref, be0_ref, w1_ref,
             out_ref):
    alo = alo_ref[...]
    ahi = ahi_ref[...]
    h0 = jnp.concatenate([alo[0] + alo[1], ahi[0] + ahi[1]], axis=1)
    dis = _dis_of(degp_ref[...])
    z = jnp.maximum(dis * h0 + b0_ref[...], 0.0)
    zb = z * (g0_ref[...] * BN_S) + be0_ref[...]
    h1 = jnp.dot(zb, w1_ref[...], preferred_element_type=jnp.float32)
    out_ref[...] = h1 * dis


def _tc_mid(alo, ahi, degp, b0, g0, be0, w1):
    grid = (N_PAD // BLK,)
    return pl.pallas_call(
        _t2_body,
        grid=grid,
        in_specs=[
            pl.BlockSpec((2, BLK, 128), lambda i: (0, i, 0)),
            pl.BlockSpec((2, BLK, 128), lambda i: (0, i, 0)),
            pl.BlockSpec((2, BLK, 128), lambda i: (0, i, 0)),
            _full((1, 256)), _full((1, 256)), _full((1, 256)),
            _full((256, 128)),
        ],
        out_specs=pl.BlockSpec((BLK, 128), lambda i: (i, 0)),
        out_shape=jax.ShapeDtypeStruct((N_PAD, 128), jnp.float32),
    )(alo, ahi, degp, b0, g0, be0, w1)


def _t3_body(a_ref, degp_ref, b1_ref, g1_ref, be1_ref, out_ref):
    a = a_ref[...]
    dis = _dis_of(degp_ref[...])
    h = dis * (a[0] + a[1]) + b1_ref[...]
    out_ref[...] = h * (g1_ref[...] * BN_S) + be1_ref[...]


def _tc_post(acc, degp, b1, g1, be1):
    grid = (N_PAD // BLK,)
    return pl.pallas_call(
        _t3_body,
        grid=grid,
        in_specs=[
            pl.BlockSpec((2, BLK, 128), lambda i: (0, i, 0)),
            pl.BlockSpec((2, BLK, 128), lambda i: (0, i, 0)),
            _full((1, 128)), _full((1, 128)), _full((1, 128)),
        ],
        out_specs=pl.BlockSpec((BLK, 128), lambda i: (i, 0)),
        out_shape=jax.ShapeDtypeStruct((N_PAD, 128), jnp.float32),
    )(acc, degp, b1, g1, be1)


def _t4_body(n_valid, hl_ref, hg_ref, mw0_ref, mb0_ref, mw1_ref, mb1_ref,
             cw0_ref, cb0_ref, cw1_ref, cb1_ref, zl_ref, zg_ref, pred_ref,
             sl_ref, sg_ref):
    i = pl.program_id(0)

    def mlp(h):
        a = jnp.maximum(
            jnp.dot(h, mw0_ref[...], preferred_element_type=jnp.float32)
            + mb0_ref[...], 0.0)
        return jnp.dot(a, mw1_ref[...],
                       preferred_element_type=jnp.float32) + mb1_ref[...]

    zl = mlp(hl_ref[...])
    zg = mlp(hg_ref[...])
    zl_ref[...] = zl
    zg_ref[...] = zg
    s = zl + zg
    a = jnp.maximum(
        jnp.dot(s, cw0_ref[...], preferred_element_type=jnp.float32)
        + cb0_ref[...], 0.0)
    pred_ref[...] = jnp.dot(a, cw1_ref[...],
                            preferred_element_type=jnp.float32) + cb1_ref[...]

    rows = lax.broadcasted_iota(jnp.int32, (BLK, 1), 0) + i * BLK
    m = (rows < n_valid).astype(jnp.float32)

    @pl.when(i == 0)
    def _():
        sl_ref[...] = jnp.zeros_like(sl_ref)
        sg_ref[...] = jnp.zeros_like(sg_ref)

    sl_ref[...] += jnp.sum(zl * m, axis=0, keepdims=True)
    sg_ref[...] += jnp.sum(zg * m, axis=0, keepdims=True)


def _tc_heads(n_valid, hl, hg, mw0, mb0, mw1, mb1, cw0, cb0, cw1, cb1):
    grid = (N_PAD // BLK,)
    return pl.pallas_call(
        functools.partial(_t4_body, n_valid),
        grid=grid,
        in_specs=[
            pl.BlockSpec((BLK, 128), lambda i: (i, 0)),
            pl.BlockSpec((BLK, 128), lambda i: (i, 0)),
            _full((128, 128)), _full((1, 128)),
            _full((128, 128)), _full((1, 128)),
            _full((128, 64)), _full((1, 64)),
            _full((64, 16)), _full((1, 16)),
        ],
        out_specs=[
            pl.BlockSpec((BLK, 128), lambda i: (i, 0)),
            pl.BlockSpec((BLK, 128), lambda i: (i, 0)),
            pl.BlockSpec((BLK, 16), lambda i: (i, 0)),
            pl.BlockSpec((1, 128), lambda i: (0, 0)),
            pl.BlockSpec((1, 128), lambda i: (0, 0)),
        ],
        out_shape=[
            jax.ShapeDtypeStruct((N_PAD, 128), jnp.float32),
            jax.ShapeDtypeStruct((N_PAD, 128), jnp.float32),
            jax.ShapeDtypeStruct((N_PAD, 16), jnp.float32),
            jax.ShapeDtypeStruct((1, 128), jnp.float32),
            jax.ShapeDtypeStruct((1, 128), jnp.float32),
        ],
    )(hl, hg, mw0, mb0, mw1, mb1, cw0, cb0, cw1, cb1)


def _t5_body(s_ref, mw0_ref, mb0_ref, mw1_ref, mb1_ref, out_ref):
    a = jnp.maximum(
        jnp.dot(s_ref[...], mw0_ref[...], preferred_element_type=jnp.float32)
        + mb0_ref[...], 0.0)
    out_ref[...] = jnp.dot(a, mw1_ref[...],
                           preferred_element_type=jnp.float32) + mb1_ref[...]


def _tc_graph_mlp(s, mw0, mb0, mw1, mb1):
    return pl.pallas_call(
        _t5_body,
        in_specs=[_full((8, 128)), _full((128, 128)), _full((1, 128)),
                  _full((128, 128)), _full((1, 128))],
        out_specs=_full((8, 128)),
        out_shape=jax.ShapeDtypeStruct((8, 128), jnp.float32),
    )(s, mw0, mb0, mw1, mb1)


def _prep_edges(ei, k):
    e = ei.shape[1]
    e_pad = NW * CHUNK * k
    src = jnp.concatenate(
        [ei[0], jnp.zeros((e_pad - e,), jnp.int32)]).reshape(NW * k, CHUNK)
    dst = jnp.concatenate(
        [ei[1], jnp.full((e_pad - e,), N_PAD - 1, jnp.int32)]
    ).reshape(NW * k, CHUNK)
    return src, dst


def kernel(x, edge_index_local, edge_index_global, lW0, lb0, lW1, lb1, lg0,
           lbe0, lg1, lbe1, gW0, gb0, gW1, gb1, gg0, gbe0, gg1, gbe1, mW0,
           mb0, mW1, mb1, cW0, cb0, cW1, cb1):
    n = x.shape[0]
    e = edge_index_local.shape[1]
    k = -(-e // (NW * CHUNK))
    k = -(-k // 8) * 8

    x_pad = jnp.pad(x, ((0, N_PAD - n), (0, 0)))
    row = lambda v: v.reshape(1, -1)

    outs = {}
    for p, ei, W0, b0, W1, b1, g0, be0, g1, be1 in (
        ("l", edge_index_local, lW0, lb0, lW1, lb1, lg0, lbe0, lg1, lbe1),
        ("g", edge_index_global, gW0, gb0, gW1, gb1, gg0, gbe0, gg1, gbe1),
    ):
        srcw, dstw = _prep_edges(ei, k)
        degp = _sc_deg(dstw, k).reshape(2, N_PAD, 128)
        hs_lo, hs_hi = _tc_scale_mm(x_pad, W0, degp)
        a_lo = _sc_seg_sum(hs_lo, srcw, dstw, k).reshape(2, N_PAD, 128)
        a_hi = _sc_seg_sum(hs_hi, srcw, dstw, k).reshape(2, N_PAD, 128)
        hs1 = _tc_mid(a_lo, a_hi, degp, row(b0), row(g0), row(be0), W1)
        a1 = _sc_seg_sum(hs1, srcw, dstw, k).reshape(2, N_PAD, 128)
        outs[p] = _tc_post(a1, degp, row(b1), row(g1), row(be1))

    zl, zg, pred, sl, sg = _tc_heads(
        n, outs["l"], outs["g"], mW0, row(mb0), mW1, row(mb1), cW0, row(cb0),
        cW1, row(cb1))
    s8 = jnp.concatenate([sl, sg, jnp.zeros((6, 128), jnp.float32)], axis=0)
    gs = _tc_graph_mlp(s8, mW0, row(mb0), mW1, row(mb1))
    return (zl[:n], zg[:n], gs[0:1], gs[1:2], pred[:n])


# speedup vs baseline: 16.7790x; 3.3507x over previous
import functools

import jax
import jax.numpy as jnp
from jax import lax
from jax.experimental import pallas as pl
from jax.experimental.pallas import tpu as pltpu
from jax.experimental.pallas import tpu_sc as plsc

N_PAD = 10240
BLK = 1024
NW = 32
CHUNK = 128
BN_S = float(1.0 / (1.0 + 1e-5) ** 0.5)

_MESH = dict(core_axis_name="c", subcore_axis_name="s")


def _fill(ref, val):
    @pl.loop(0, CHUNK)
    def _(r):
        for cc in range(8):
            ref[r, pl.ds(cc * 16, 16)] = jnp.full((16,), val, jnp.float32)


def _zero_acc(buf_v, acc_sh, base):
    _fill(buf_v, 0.0)
    for jj in range(N_PAD // 16 // 128):
        pltpu.sync_copy(buf_v, acc_sh.at[pl.ds(base + jj * 128, 128)])


def _write_out(acc_sh, out_hbm, cid, base):
    for jj in range(N_PAD // 16 // 128):
        pltpu.sync_copy(
            acc_sh.at[pl.ds(base + jj * 128, 128)],
            out_hbm.at[pl.ds(cid * N_PAD + base + jj * 128, 128)])


def _sc_seg_sum(table, srcw, dstw, k):
    rpt = N_PAD // 16
    kh = k // 2

    @functools.partial(
        pl.kernel,
        out_type=jax.ShapeDtypeStruct((2 * N_PAD, 128), jnp.float32),
        mesh=plsc.VectorSubcoreMesh(**_MESH),
        scratch_types=[
            pltpu.VMEM((kh, CHUNK), jnp.int32),
            pltpu.VMEM((kh, CHUNK), jnp.int32),
            pltpu.VMEM((CHUNK, 128), jnp.float32),
            pltpu.VMEM((CHUNK, 128), jnp.float32),
            pltpu.VMEM_SHARED((N_PAD, 128), jnp.float32),
            pltpu.SemaphoreType.DMA,
            pltpu.SemaphoreType.DMA,
            pltpu.SemaphoreType.DMA,
            pltpu.SemaphoreType.DMA,
        ],
    )
    def seg(table_hbm, src_hbm, dst_hbm, out_hbm, src_v, dst_v, buf0, buf1,
            acc_sh, g0, g1, s0, s1):
        cid = lax.axis_index("c")
        sid = lax.axis_index("s")
        wid = cid * 16 + sid
        bufs, gsem, ssem = (buf0, buf1), (g0, g1), (s0, s1)

        def wait(b, sem):
            pltpu.make_async_copy(table_hbm.at[pl.ds(0, CHUNK)], bufs[b],
                                  sem[b]).wait()

        base = sid * rpt
        _zero_acc(buf0, acc_sh, base)
        plsc.subcore_barrier()

        for ph in range(2):
            off = wid * k + ph * kh
            pltpu.sync_copy(src_hbm.at[pl.ds(off, kh)], src_v)
            pltpu.sync_copy(dst_hbm.at[pl.ds(off, kh)], dst_v)
            for b in range(2):
                pltpu.async_copy(table_hbm.at[src_v.at[b]], bufs[b], gsem[b])

            @pl.loop(0, kh - 2, step=2)
            def _(j):
                for b in range(2):
                    jj = j + b
                    wait(b, gsem)
                    pltpu.async_copy(bufs[b], acc_sh.at[dst_v.at[jj]],
                                     ssem[b], add=True)
                    wait(b, ssem)
                    pltpu.async_copy(table_hbm.at[src_v.at[jj + 2]], bufs[b],
                                     gsem[b])

            for b in range(2):
                wait(b, gsem)
                pltpu.sync_copy(bufs[b], acc_sh.at[dst_v.at[kh - 2 + b]],
                                add=True)

        plsc.subcore_barrier()
        _write_out(acc_sh, out_hbm, cid, base)

    return seg(table, srcw, dstw)


def _sc_deg(dstw, k):
    rpt = N_PAD // 16

    @functools.partial(
        pl.kernel,
        out_type=jax.ShapeDtypeStruct((2 * N_PAD, 128), jnp.float32),
        mesh=plsc.VectorSubcoreMesh(**_MESH),
        scratch_types=[
            pltpu.VMEM((k, CHUNK), jnp.int32),
            pltpu.VMEM((CHUNK, 128), jnp.float32),
            pltpu.VMEM_SHARED((N_PAD, 128), jnp.float32),
            pltpu.SemaphoreType.DMA,
        ],
    )
    def deg(dst_hbm, out_hbm, dst_v, ones_v, acc_sh, sem):
        cid = lax.axis_index("c")
        sid = lax.axis_index("s")
        wid = cid * 16 + sid
        base = sid * rpt

        _zero_acc(ones_v, acc_sh, base)
        _fill(ones_v, 1.0)
        plsc.subcore_barrier()

        pltpu.sync_copy(dst_hbm.at[pl.ds(wid * k, k)], dst_v)

        @pl.loop(0, k, step=8)
        def _(j):
            for b in range(8):
                pltpu.async_copy(ones_v, acc_sh.at[dst_v.at[j + b]], sem,
                                 add=True)
            for b in range(8):
                pltpu.make_async_copy(out_hbm.at[pl.ds(0, CHUNK)], ones_v,
                                      sem).wait()

        plsc.subcore_barrier()
        _write_out(acc_sh, out_hbm, cid, base)

    return deg(dstw)


def _dis_of(degp):
    deg = degp[0] + degp[1]
    return jnp.where(deg > 0.0, lax.rsqrt(jnp.maximum(deg, 1e-30)),
                     0.0)[:, 0:1]


def _full(shape):
    return pl.BlockSpec(shape, lambda *_: tuple(0 for _ in shape))


def _t1_body(x_ref, w_ref, degp_ref, lo_ref, hi_ref):
    h = jnp.dot(x_ref[...], w_ref[...], preferred_element_type=jnp.float32)
    dis = _dis_of(degp_ref[...])
    hs = h * dis
    lo_ref[...] = hs[:, :128]
    hi_ref[...] = hs[:, 128:]


def _tc_scale_mm(x, w, degp):
    grid = (N_PAD // BLK,)
    out = pl.pallas_call(
        _t1_body,
        grid=grid,
        in_specs=[
            pl.BlockSpec((BLK, 128), lambda i: (i, 0)),
            _full(w.shape),
            pl.BlockSpec((2, BLK, 128), lambda i: (0, i, 0)),
        ],
        out_specs=[pl.BlockSpec((BLK, 128), lambda i: (i, 0))] * 2,
        out_shape=[jax.ShapeDtypeStruct((N_PAD, 128), jnp.float32)] * 2,
    )(x, w, degp)
    return out[0], out[1]


def _t2_body(alo_ref, ahi_ref, degp_ref, b0_ref, g0_ref, be0_ref, w1_ref,
             out_ref):
    alo = alo_ref[...]
    ahi = ahi_ref[...]
    h0 = jnp.concatenate([alo[0] + alo[1], ahi[0] + ahi[1]], axis=1)
    dis = _dis_of(degp_ref[...])
    z = jnp.maximum(dis * h0 + b0_ref[...], 0.0)
    zb = z * (g0_ref[...] * BN_S) + be0_ref[...]
    h1 = jnp.dot(zb, w1_ref[...], preferred_element_type=jnp.float32)
    out_ref[...] = h1 * dis


def _tc_mid(alo, ahi, degp, b0, g0, be0, w1):
    grid = (N_PAD // BLK,)
    return pl.pallas_call(
        _t2_body,
        grid=grid,
        in_specs=[
            pl.BlockSpec((2, BLK, 128), lambda i: (0, i, 0)),
            pl.BlockSpec((2, BLK, 128), lambda i: (0, i, 0)),
            pl.BlockSpec((2, BLK, 128), lambda i: (0, i, 0)),
            _full((1, 256)), _full((1, 256)), _full((1, 256)),
            _full((256, 128)),
        ],
        out_specs=pl.BlockSpec((BLK, 128), lambda i: (i, 0)),
        out_shape=jax.ShapeDtypeStruct((N_PAD, 128), jnp.float32),
    )(alo, ahi, degp, b0, g0, be0, w1)


def _t3_body(a_ref, degp_ref, b1_ref, g1_ref, be1_ref, out_ref):
    a = a_ref[...]
    dis = _dis_of(degp_ref[...])
    h = dis * (a[0] + a[1]) + b1_ref[...]
    out_ref[...] = h * (g1_ref[...] * BN_S) + be1_ref[...]


def _tc_post(acc, degp, b1, g1, be1):
    grid = (N_PAD // BLK,)
    return pl.pallas_call(
        _t3_body,
        grid=grid,
        in_specs=[
            pl.BlockSpec((2, BLK, 128), lambda i: (0, i, 0)),
            pl.BlockSpec((2, BLK, 128), lambda i: (0, i, 0)),
            _full((1, 128)), _full((1, 128)), _full((1, 128)),
        ],
        out_specs=pl.BlockSpec((BLK, 128), lambda i: (i, 0)),
        out_shape=jax.ShapeDtypeStruct((N_PAD, 128), jnp.float32),
    )(acc, degp, b1, g1, be1)


def _t4_body(n_valid, hl_ref, hg_ref, mw0_ref, mb0_ref, mw1_ref, mb1_ref,
             cw0_ref, cb0_ref, cw1_ref, cb1_ref, zl_ref, zg_ref, pred_ref,
             sl_ref, sg_ref):
    i = pl.program_id(0)

    def mlp(h):
        a = jnp.maximum(
            jnp.dot(h, mw0_ref[...], preferred_element_type=jnp.float32)
            + mb0_ref[...], 0.0)
        return jnp.dot(a, mw1_ref[...],
                       preferred_element_type=jnp.float32) + mb1_ref[...]

    zl = mlp(hl_ref[...])
    zg = mlp(hg_ref[...])
    zl_ref[...] = zl
    zg_ref[...] = zg
    s = zl + zg
    a = jnp.maximum(
        jnp.dot(s, cw0_ref[...], preferred_element_type=jnp.float32)
        + cb0_ref[...], 0.0)
    pred_ref[...] = jnp.dot(a, cw1_ref[...],
                            preferred_element_type=jnp.float32) + cb1_ref[...]

    rows = lax.broadcasted_iota(jnp.int32, (BLK, 1), 0) + i * BLK
    m = (rows < n_valid).astype(jnp.float32)

    @pl.when(i == 0)
    def _():
        sl_ref[...] = jnp.zeros_like(sl_ref)
        sg_ref[...] = jnp.zeros_like(sg_ref)

    sl_ref[...] += jnp.sum(zl * m, axis=0, keepdims=True)
    sg_ref[...] += jnp.sum(zg * m, axis=0, keepdims=True)


def _tc_heads(n_valid, hl, hg, mw0, mb0, mw1, mb1, cw0, cb0, cw1, cb1):
    grid = (N_PAD // BLK,)
    return pl.pallas_call(
        functools.partial(_t4_body, n_valid),
        grid=grid,
        in_specs=[
            pl.BlockSpec((BLK, 128), lambda i: (i, 0)),
            pl.BlockSpec((BLK, 128), lambda i: (i, 0)),
            _full((128, 128)), _full((1, 128)),
            _full((128, 128)), _full((1, 128)),
            _full((128, 64)), _full((1, 64)),
            _full((64, 16)), _full((1, 16)),
        ],
        out_specs=[
            pl.BlockSpec((BLK, 128), lambda i: (i, 0)),
            pl.BlockSpec((BLK, 128), lambda i: (i, 0)),
            pl.BlockSpec((BLK, 16), lambda i: (i, 0)),
            pl.BlockSpec((1, 128), lambda i: (0, 0)),
            pl.BlockSpec((1, 128), lambda i: (0, 0)),
        ],
        out_shape=[
            jax.ShapeDtypeStruct((N_PAD, 128), jnp.float32),
            jax.ShapeDtypeStruct((N_PAD, 128), jnp.float32),
            jax.ShapeDtypeStruct((N_PAD, 16), jnp.float32),
            jax.ShapeDtypeStruct((1, 128), jnp.float32),
            jax.ShapeDtypeStruct((1, 128), jnp.float32),
        ],
    )(hl, hg, mw0, mb0, mw1, mb1, cw0, cb0, cw1, cb1)


def _t5_body(s_ref, mw0_ref, mb0_ref, mw1_ref, mb1_ref, out_ref):
    a = jnp.maximum(
        jnp.dot(s_ref[...], mw0_ref[...], preferred_element_type=jnp.float32)
        + mb0_ref[...], 0.0)
    out_ref[...] = jnp.dot(a, mw1_ref[...],
                           preferred_element_type=jnp.float32) + mb1_ref[...]


def _tc_graph_mlp(s, mw0, mb0, mw1, mb1):
    return pl.pallas_call(
        _t5_body,
        in_specs=[_full((8, 128)), _full((128, 128)), _full((1, 128)),
                  _full((128, 128)), _full((1, 128))],
        out_specs=_full((8, 128)),
        out_shape=jax.ShapeDtypeStruct((8, 128), jnp.float32),
    )(s, mw0, mb0, mw1, mb1)


def _prep_edges(ei, k, n):
    e = ei.shape[1]
    pad = NW * CHUNK * k - e
    ar = jnp.arange(pad, dtype=jnp.int32)
    src = jnp.concatenate([ei[0], ar % n]).reshape(NW * k, CHUNK)
    dst = jnp.concatenate(
        [ei[1], n + (ar % (N_PAD - n))]).reshape(NW * k, CHUNK)
    return src, dst


def kernel(x, edge_index_local, edge_index_global, lW0, lb0, lW1, lb1, lg0,
           lbe0, lg1, lbe1, gW0, gb0, gW1, gb1, gg0, gbe0, gg1, gbe1, mW0,
           mb0, mW1, mb1, cW0, cb0, cW1, cb1):
    n = x.shape[0]
    e = edge_index_local.shape[1]
    k = -(-e // (NW * CHUNK))
    k = -(-k // 8) * 8

    x_pad = jnp.pad(x, ((0, N_PAD - n), (0, 0)))
    row = lambda v: v.reshape(1, -1)

    outs = {}
    for p, ei, W0, b0, W1, b1, g0, be0, g1, be1 in (
        ("l", edge_index_local, lW0, lb0, lW1, lb1, lg0, lbe0, lg1, lbe1),
        ("g", edge_index_global, gW0, gb0, gW1, gb1, gg0, gbe0, gg1, gbe1),
    ):
        srcw, dstw = _prep_edges(ei, k, n)
        degp = _sc_deg(dstw, k).reshape(2, N_PAD, 128)
        hs_lo, hs_hi = _tc_scale_mm(x_pad, W0, degp)
        a_lo = _sc_seg_sum(hs_lo, srcw, dstw, k).reshape(2, N_PAD, 128)
        a_hi = _sc_seg_sum(hs_hi, srcw, dstw, k).reshape(2, N_PAD, 128)
        hs1 = _tc_mid(a_lo, a_hi, degp, row(b0), row(g0), row(be0), W1)
        a1 = _sc_seg_sum(hs1, srcw, dstw, k).reshape(2, N_PAD, 128)
        outs[p] = _tc_post(a1, degp, row(b1), row(g1), row(be1))

    zl, zg, pred, sl, sg = _tc_heads(
        n, outs["l"], outs["g"], mW0, row(mb0), mW1, row(mb1), cW0, row(cb0),
        cW1, row(cb1))
    s8 = jnp.concatenate([sl, sg, jnp.zeros((6, 128), jnp.float32)], axis=0)
    gs = _tc_graph_mlp(s8, mW0, row(mb0), mW1, row(mb1))
    return (zl[:n], zg[:n], gs[0:1], gs[1:2], pred[:n])
